# baseline (speedup 1.0000x reference)
import dataclasses
import functools

import jax
import jax.numpy as jnp
from jax import lax
from jax.experimental import pallas as pl
from jax.experimental.pallas import tpu as pltpu
from jax.experimental.pallas import tpu_sc as plsc

N_LEVELS = 16
LOG2_T = 19
TABLE_SIZE = 1 << LOG2_T
MASK = TABLE_SIZE - 1
BASE_RES = 16
B_SCALE = 1.39
RES = [int(BASE_RES * (B_SCALE ** i)) for i in range(N_LEVELS)]
P1_I32 = 2654435761 - (1 << 32)
P2_I32 = 805459861

NC, NS, L = 2, 16, 16
NW = NC * NS
P = 256
CH = P // L
D = 8


def kernel(x, tables):
    n = x.shape[0]
    xt = x.reshape(-1)
    tab = (tables.reshape(N_LEVELS, TABLE_SIZE // 128, 128, 2)
           .transpose(0, 1, 3, 2)
           .reshape(N_LEVELS * TABLE_SIZE * 2 // D, D))
    pts_per_tile = n // NW
    nblk = pts_per_tile // P
    mesh = plsc.VectorSubcoreMesh(core_axis_name="c", subcore_axis_name="s")
    cp = pltpu.CompilerParams()
    if "needs_layout_passes" in pltpu.CompilerParams.__dataclass_fields__:
        cp = dataclasses.replace(cp, needs_layout_passes=False)
    if "use_tc_tiling_on_sc" in pltpu.CompilerParams.__dataclass_fields__:
        cp = dataclasses.replace(cp, use_tc_tiling_on_sc=False)

    @functools.partial(
        pl.kernel,
        mesh=mesh,
        compiler_params=cp,
        out_type=jax.ShapeDtypeStruct((n * 2 * N_LEVELS,), jnp.float32),
        scratch_types=[
            pltpu.VMEM((3 * P,), jnp.float32),
            pltpu.VMEM((3 * P,), jnp.float32),
            pltpu.VMEM((3 * P,), jnp.float32),
            pltpu.VMEM((16 * P,), jnp.int32),
            pltpu.VMEM((16 * P,), jnp.int32),
            pltpu.VMEM((8 * P,), jnp.int32),
            pltpu.VMEM((8 * P,), jnp.int32),
            pltpu.VMEM((16 * P, D), jnp.float32),
            pltpu.VMEM((16 * P, D), jnp.float32),
            pltpu.VMEM((2 * N_LEVELS * P,), jnp.float32),
            pltpu.SemaphoreType.DMA,
            pltpu.SemaphoreType.DMA,
        ],
    )
    def sc_kernel(xt_hbm, tab_hbm, out_hbm, xv, wv0, wv1, idxv0, idxv1,
                  offv0, offv1, rowsv0, rowsv1, outv, sem0, sem1):
        wid = lax.axis_index("s") * NC + lax.axis_index("c")
        iota = lax.iota(jnp.int32, L)
        bufs = [(wv0, idxv0, offv0, rowsv0, sem0),
                (wv1, idxv1, offv1, rowsv1, sem1)]

        def hash_phase(lvl):
            res = float(RES[lvl])
            base17 = lvl << 17
            wv, idxv, offv, _, _ = bufs[lvl % 2]

            @pl.loop(0, CH)
            def _hash(ch):
                p0 = ch * L
                pvec = p0 + iota
                v3p = pvec * 3
                v8p = pvec * 8
                vi = []
                for d in range(3):
                    xf = plsc.load_gather(xv, [v3p + d]) * res
                    vid = xf.astype(jnp.int32)
                    wv[pl.ds(d * P + p0, L)] = xf - vid.astype(jnp.float32)
                    vi.append(vid)
                hx0 = vi[0]
                hx1 = vi[0] + 1
                hy0 = vi[1] * P1_I32
                hy1 = hy0 + P1_I32
                hz0 = vi[2] * P2_I32
                hz1 = hz0 + P2_I32
                a = [hx0 ^ hy0, hx0 ^ hy1, hx1 ^ hy0, hx1 ^ hy1]
                for c in range(8):
                    i, j, k = c >> 2, (c >> 1) & 1, c & 1
                    h = (a[2 * i + j] ^ (hz1 if k else hz0)) & MASK
                    r0 = (h >> 3) + ((h >> 7) << 4) + base17
                    idxv[pl.ds(c * P + p0, L)] = r0
                    idxv[pl.ds(8 * P + c * P + p0, L)] = r0 + 16
                    offv[pl.ds(c * P + p0, L)] = v8p + (h & 7)

        def start_gather(lvl):
            _, idxv, _, rowsv, sem = bufs[lvl % 2]
            return pltpu.async_copy(tab_hbm.at[idxv], rowsv, sem)

        def interp_phase(lvl):
            wv, _, offv, rowsv, _ = bufs[lvl % 2]

            @pl.loop(0, CH)
            def _interp(ch):
                p0 = ch * L
                pvec = p0 + iota
                v32p = pvec * (2 * N_LEVELS)
                w0 = wv[pl.ds(p0, L)]
                w1 = wv[pl.ds(P + p0, L)]
                w2 = wv[pl.ds(2 * P + p0, L)]
                u0 = 1.0 - w0
                u1 = 1.0 - w1
                u2 = 1.0 - w2
                yz = [u1 * u2, u1 * w2, w1 * u2, w1 * w2]
                w8 = [u0 * yz[0], u0 * yz[1], u0 * yz[2], u0 * yz[3],
                      w0 * yz[0], w0 * yz[1], w0 * yz[2], w0 * yz[3]]
                zs = jnp.zeros((L,), jnp.int32)
                acc = [None, None]
                for c in range(8):
                    fbase = offv[pl.ds(c * P + p0, L)]
                    for f in range(2):
                        flat = fbase + (f * 64 * P + 8 * c * P)
                        v = plsc.load_gather(rowsv, [zs, flat])
                        term = v * w8[c]
                        acc[f] = term if acc[f] is None else acc[f] + term
                for f in range(2):
                    plsc.store_scatter(outv, [v32p + (2 * lvl + f)], acc[f])

        @pl.loop(0, nblk)
        def _blk(blk):
            gbase = wid * pts_per_tile + blk * P
            pltpu.sync_copy(xt_hbm.at[pl.ds(3 * gbase, 3 * P)], xv)

            hash_phase(0)
            handles = {0: start_gather(0)}
            for lvl in range(N_LEVELS):
                if lvl + 1 < N_LEVELS:
                    hash_phase(lvl + 1)
                    handles[lvl + 1] = start_gather(lvl + 1)
                handles[lvl].wait()
                interp_phase(lvl)

            pltpu.sync_copy(outv, out_hbm.at[pl.ds(gbase * 2 * N_LEVELS,
                                                   2 * N_LEVELS * P)])

    return sc_kernel(xt, tab).reshape(n, 2 * N_LEVELS)


# speedup vs baseline: 1.7038x; 1.7038x over previous
import dataclasses
import functools

import jax
import jax.numpy as jnp
from jax import lax
from jax.experimental import pallas as pl
from jax.experimental.pallas import tpu as pltpu
from jax.experimental.pallas import tpu_sc as plsc

N_LEVELS = 16
LOG2_T = 19
TABLE_SIZE = 1 << LOG2_T
MASK = TABLE_SIZE - 1
BASE_RES = 16
B_SCALE = 1.39
RES = [int(BASE_RES * (B_SCALE ** i)) for i in range(N_LEVELS)]
P1_I32 = 2654435761 - (1 << 32)
P2_I32 = 805459861

NC, NS, L = 2, 16, 16
NW = NC * NS
P = 512
CH = P // L
D = 8
NROWS = N_LEVELS * TABLE_SIZE * 2 // D
NRC = 4096


def _mk_compiler_params():
    cp = pltpu.CompilerParams()
    if "needs_layout_passes" in pltpu.CompilerParams.__dataclass_fields__:
        cp = dataclasses.replace(cp, needs_layout_passes=False)
    if "use_tc_tiling_on_sc" in pltpu.CompilerParams.__dataclass_fields__:
        cp = dataclasses.replace(cp, use_tc_tiling_on_sc=False)
    return cp


def kernel(x, tables):
    n = x.shape[0]
    xt = x.reshape(-1)
    tab = (tables.reshape(N_LEVELS, TABLE_SIZE // 128, 128, 2)
           .transpose(0, 1, 3, 2)
           .reshape(NROWS, D))
    pts_per_tile = n // NW
    nblk = pts_per_tile // P
    mesh = plsc.VectorSubcoreMesh(core_axis_name="c", subcore_axis_name="s")
    cp = _mk_compiler_params()

    @functools.partial(
        pl.kernel,
        mesh=mesh,
        compiler_params=cp,
        out_type=jax.ShapeDtypeStruct((NROWS, D), jnp.float32),
        scratch_types=[
            pltpu.VMEM((NRC, D), jnp.float32),
            pltpu.VMEM((NRC, D), jnp.float32),
        ],
    )
    def convert_kernel(tabres_hbm, tabrm_hbm, inv, outv):
        wid = lax.axis_index("s") * NC + lax.axis_index("c")
        iota = lax.iota(jnp.int32, L)
        zs = jnp.zeros((L,), jnp.int32)
        rows_per_tile = NROWS // NW
        nch = rows_per_tile // NRC

        @pl.loop(0, nch)
        def _chunk(ci):
            r0 = wid * rows_per_tile + ci * NRC
            pltpu.sync_copy(tabres_hbm.at[pl.ds(r0, NRC)], inv)

            @pl.loop(0, NRC * D // 256)
            def _grp(b):
                wbase = b * 256
                for i0 in range(0, 128, L):
                    src = wbase + i0 + iota
                    f0 = plsc.load_gather(inv, [zs, src])
                    f1 = plsc.load_gather(inv, [zs, src + 128])
                    dst = wbase + 2 * (i0 + iota)
                    plsc.store_scatter(outv, [zs, dst], f0)
                    plsc.store_scatter(outv, [zs, dst + 1], f1)

            pltpu.sync_copy(outv, tabrm_hbm.at[pl.ds(r0, NRC)])

    @functools.partial(
        pl.kernel,
        mesh=mesh,
        compiler_params=cp,
        out_type=jax.ShapeDtypeStruct((n * 2 * N_LEVELS,), jnp.float32),
        scratch_types=[
            pltpu.VMEM((3 * P,), jnp.float32),
            pltpu.VMEM((3 * P,), jnp.float32),
            pltpu.VMEM((3 * P,), jnp.float32),
            pltpu.VMEM((8 * P,), jnp.int32),
            pltpu.VMEM((8 * P,), jnp.int32),
            pltpu.VMEM((8 * P,), jnp.int32),
            pltpu.VMEM((8 * P,), jnp.int32),
            pltpu.VMEM((8 * P, D), jnp.float32),
            pltpu.VMEM((8 * P, D), jnp.float32),
            pltpu.VMEM((2 * N_LEVELS * P,), jnp.float32),
            pltpu.SemaphoreType.DMA,
            pltpu.SemaphoreType.DMA,
        ],
    )
    def sc_kernel(xt_hbm, tab_hbm, out_hbm, xv, wv0, wv1, idxv0, idxv1,
                  offv0, offv1, rowsv0, rowsv1, outv, sem0, sem1):
        wid = lax.axis_index("s") * NC + lax.axis_index("c")
        iota = lax.iota(jnp.int32, L)
        bufs = [(wv0, idxv0, offv0, rowsv0, sem0),
                (wv1, idxv1, offv1, rowsv1, sem1)]

        def hash_phase(lvl):
            res = float(RES[lvl])
            base17 = lvl << 17
            wv, idxv, offv, _, _ = bufs[lvl % 2]

            @pl.loop(0, CH)
            def _hash(ch):
                p0 = ch * L
                pvec = p0 + iota
                v3p = pvec * 3
                v8p = pvec * 8
                vi = []
                for d in range(3):
                    xf = plsc.load_gather(xv, [v3p + d]) * res
                    vid = xf.astype(jnp.int32)
                    wv[pl.ds(d * P + p0, L)] = xf - vid.astype(jnp.float32)
                    vi.append(vid)
                hx0 = vi[0]
                hx1 = vi[0] + 1
                hy0 = vi[1] * P1_I32
                hy1 = hy0 + P1_I32
                hz0 = vi[2] * P2_I32
                hz1 = hz0 + P2_I32
                a = [hx0 ^ hy0, hx0 ^ hy1, hx1 ^ hy0, hx1 ^ hy1]
                for c in range(8):
                    i, j, k = c >> 2, (c >> 1) & 1, c & 1
                    h = (a[2 * i + j] ^ (hz1 if k else hz0)) & MASK
                    idxv[pl.ds(c * P + p0, L)] = (h >> 2) + base17
                    offv[pl.ds(c * P + p0, L)] = v8p + ((h & 3) << 1)

        def start_gather(lvl):
            _, idxv, _, rowsv, sem = bufs[lvl % 2]
            return pltpu.async_copy(tab_hbm.at[idxv], rowsv, sem)

        def interp_phase(lvl):
            wv, _, offv, rowsv, _ = bufs[lvl % 2]

            @pl.loop(0, CH)
            def _interp(ch):
                p0 = ch * L
                pvec = p0 + iota
                v32p = pvec * (2 * N_LEVELS)
                w0 = wv[pl.ds(p0, L)]
                w1 = wv[pl.ds(P + p0, L)]
                w2 = wv[pl.ds(2 * P + p0, L)]
                u0 = 1.0 - w0
                u1 = 1.0 - w1
                u2 = 1.0 - w2
                yz = [u1 * u2, u1 * w2, w1 * u2, w1 * w2]
                w8 = [u0 * yz[0], u0 * yz[1], u0 * yz[2], u0 * yz[3],
                      w0 * yz[0], w0 * yz[1], w0 * yz[2], w0 * yz[3]]
                zs = jnp.zeros((L,), jnp.int32)
                acc = [None, None]
                for c in range(8):
                    fbase = offv[pl.ds(c * P + p0, L)]
                    for f in range(2):
                        flat = fbase + (8 * c * P + f)
                        v = plsc.load_gather(rowsv, [zs, flat])
                        term = v * w8[c]
                        acc[f] = term if acc[f] is None else acc[f] + term
                for f in range(2):
                    plsc.store_scatter(outv, [v32p + (2 * lvl + f)], acc[f])

        @pl.loop(0, nblk)
        def _blk(blk):
            gbase = wid * pts_per_tile + blk * P
            pltpu.sync_copy(xt_hbm.at[pl.ds(3 * gbase, 3 * P)], xv)

            hash_phase(0)
            handles = {0: start_gather(0)}
            for lvl in range(N_LEVELS):
                if lvl + 1 < N_LEVELS:
                    hash_phase(lvl + 1)
                    handles[lvl + 1] = start_gather(lvl + 1)
                handles[lvl].wait()
                interp_phase(lvl)

            pltpu.sync_copy(outv, out_hbm.at[pl.ds(gbase * 2 * N_LEVELS,
                                                   2 * N_LEVELS * P)])

    tabrm = convert_kernel(tab)
    return sc_kernel(xt, tabrm).reshape(n, 2 * N_LEVELS)
